# baseline (speedup 1.0000x reference)
import functools

import jax
import jax.numpy as jnp
from jax import lax
from jax.experimental import pallas as pl
from jax.experimental.pallas import tpu as pltpu
from jax.experimental.pallas import tpu_sc as plsc

CH = 128
L = 16
BL = 32
NC = 2
NS = 16
NW = NC * NS
EB = 128
N_PAD = 10240
ROWS_PER_TILE = N_PAD // NS
CNT_W = 16
NREL_PAD = 16
RING = 3
IR = 2 * RING


def _make_sc_agg(nb):
    assert nb % IR == 0
    out_type = [
        jax.ShapeDtypeStruct((NC, N_PAD, CH), jnp.bfloat16),
        jax.ShapeDtypeStruct((NC, N_PAD, CNT_W), jnp.float32),
    ]
    scratch = [
        pltpu.VMEM((IR, 3, EB), jnp.int32),
        pltpu.VMEM((RING, EB, CH), jnp.bfloat16),
        pltpu.VMEM((RING, EB, CH), jnp.bfloat16),
        pltpu.VMEM((NREL_PAD, CH), jnp.bfloat16),
        pltpu.VMEM((EB, CNT_W), jnp.float32),
        pltpu.VMEM_SHARED((N_PAD, CH), jnp.bfloat16),
        pltpu.VMEM_SHARED((N_PAD, CNT_W), jnp.float32),
    ]
    scratch += [pltpu.SemaphoreType.DMA] * RING
    scratch += [pltpu.SemaphoreType.DMA] * IR
    mesh = plsc.VectorSubcoreMesh(core_axis_name="c", subcore_axis_name="s")

    @functools.partial(pl.kernel, out_type=tuple(out_type), mesh=mesh,
                       scratch_types=scratch,
                       compiler_params=pltpu.CompilerParams(
                           use_tc_tiling_on_sc=False))
    def sc_agg(ent_hbm, rel_hbm, edata_hbm, zrow_hbm, zcnt_hbm, ones_hbm,
               *rest):
        (part_hbm, cnt_hbm, eslot, erow, msg, reltab, ones_v, acc, cacc,
         *sems) = rest
        sem_g = sems[:RING]
        sem_i = sems[RING:]
        c = lax.axis_index("c")
        s = lax.axis_index("s")
        wid = c * NS + s
        pltpu.sync_copy(rel_hbm, reltab)
        tile_rows = pl.ds(s * ROWS_PER_TILE, ROWS_PER_TILE)
        pltpu.sync_copy(zrow_hbm, acc.at[tile_rows])
        pltpu.sync_copy(zcnt_hbm, cacc.at[tile_rows])
        pltpu.sync_copy(ones_hbm, ones_v)
        plsc.subcore_barrier()

        def fire_idx(u, b):
            pltpu.async_copy(edata_hbm.at[wid, b], eslot.at[u], sem_i[u])

        def wait_idx(u):
            pltpu.make_async_copy(edata_hbm.at[0, 0], eslot.at[u],
                                  sem_i[u]).wait()

        def fire_row(k, u):
            pltpu.async_copy(ent_hbm.at[eslot.at[u, 0]], erow.at[k],
                             sem_g[k])

        def wait_row(k):
            pltpu.make_async_copy(ent_hbm.at[eslot.at[0, 0]],
                                  erow.at[k], sem_g[k]).wait()

        for u in range(IR):
            fire_idx(u, u)
        for k in range(RING):
            wait_idx(k)
            fire_row(k, k)

        def group_body(g, carry):
            for u in range(IR):
                k = u % RING
                b = g * IR + u
                wait_row(k)

                def mul_body(g2, carry2):
                    tvec = eslot[u, 1, pl.ds(g2 * L, L)]
                    for i in range(L):
                        e = g2 * L + i
                        t = tvec[i]
                        for j in range(CH // BL):
                            sl = pl.ds(j * BL, BL)
                            msg[k, e, sl] = erow[k, e, sl] * reltab[t, sl]
                    return carry2

                lax.fori_loop(0, EB // L, mul_body, 0)

                pltpu.sync_copy(msg.at[k], acc.at[eslot.at[u, 2]], add=True)
                pltpu.sync_copy(ones_v, cacc.at[eslot.at[u, 2]], add=True)

                @pl.when(b + IR < nb)
                def _():
                    fire_idx(u, b + IR)

                @pl.when(b + RING < nb)
                def _():
                    u2 = (u + RING) % IR
                    wait_idx(u2)
                    fire_row(k, u2)
            return carry

        lax.fori_loop(0, nb // IR, group_body, 0)
        plsc.subcore_barrier()
        pltpu.sync_copy(acc.at[tile_rows], part_hbm.at[c, tile_rows])
        pltpu.sync_copy(cacc.at[tile_rows], cnt_hbm.at[c, tile_rows])

    return sc_agg


TBLK = 1280


def _tc_hop(part, cntp, res_in):
    def body(part_ref, cntp_ref, res_ref, ent_next_ref, res_out_ref):
        ssum = (part_ref[0].astype(jnp.float32)
                + part_ref[1].astype(jnp.float32))
        cnt = cntp_ref[0, :, 0:1] + cntp_ref[1, :, 0:1]
        mean = ssum / jnp.maximum(cnt, 1.0)
        nrm = jnp.sqrt(jnp.sum(mean * mean, axis=1, keepdims=True))
        normd = mean / jnp.maximum(nrm, 1e-12)
        ent_next_ref[...] = normd.astype(jnp.bfloat16)
        res_out_ref[...] = res_ref[...] + normd

    grid = N_PAD // TBLK
    return pl.pallas_call(
        body,
        grid=(grid,),
        in_specs=[
            pl.BlockSpec((NC, TBLK, CH), lambda i: (0, i, 0)),
            pl.BlockSpec((NC, TBLK, CNT_W), lambda i: (0, i, 0)),
            pl.BlockSpec((TBLK, CH), lambda i: (i, 0)),
        ],
        out_specs=[
            pl.BlockSpec((TBLK, CH), lambda i: (i, 0)),
            pl.BlockSpec((TBLK, CH), lambda i: (i, 0)),
        ],
        out_shape=[
            jax.ShapeDtypeStruct((N_PAD, CH), jnp.bfloat16),
            jax.ShapeDtypeStruct((N_PAD, CH), jnp.float32),
        ],
    )(part, cntp, res_in)


def _tc_rel(relp):
    def body(rel_ref, reln_ref, relres_ref):
        r = rel_ref[...]
        nrm = jnp.sqrt(jnp.sum(r * r, axis=1, keepdims=True))
        rn = r / jnp.maximum(nrm, 1e-12)
        reln_ref[...] = rn.astype(jnp.bfloat16)
        relres_ref[...] = r + 2.0 * rn

    return pl.pallas_call(
        body,
        out_shape=[
            jax.ShapeDtypeStruct((NREL_PAD, CH), jnp.bfloat16),
            jax.ShapeDtypeStruct((NREL_PAD, CH), jnp.float32),
        ],
    )(relp)


def _tc_drug(res2, drug0p, ent0p):
    def body(res_ref, drug0_ref, ent0_ref, drug_ref):
        drug_ref[...] = drug0_ref[...] + (res_ref[...] - ent0_ref[...])

    grid = N_PAD // TBLK
    return pl.pallas_call(
        body,
        grid=(grid,),
        in_specs=[
            pl.BlockSpec((TBLK, CH), lambda i: (i, 0)),
            pl.BlockSpec((TBLK, CH), lambda i: (i, 0)),
            pl.BlockSpec((TBLK, CH), lambda i: (i, 0)),
        ],
        out_specs=[
            pl.BlockSpec((TBLK, CH), lambda i: (i, 0)),
        ],
        out_shape=[
            jax.ShapeDtypeStruct((N_PAD, CH), jnp.float32),
        ],
    )(res2, drug0p, ent0p)[0]


def kernel(drug_emb, entity_emb, relation_emb, edge_index, edge_type, gpu_id):
    n_ent, _ = entity_emb.shape
    n_drugs = drug_emb.shape[0]
    n_rel = relation_emb.shape[0]
    e = edge_type.shape[0]

    head = edge_index[0].astype(jnp.int32)
    tail = edge_index[1].astype(jnp.int32)
    etype = edge_type.astype(jnp.int32)

    nb = IR * -(-e // (NW * EB * IR))
    e_pad = NW * nb * EB
    pad = e_pad - e
    tail_p = jnp.concatenate([tail, jnp.zeros((pad,), jnp.int32)])
    etype_p = jnp.concatenate([etype, jnp.full((pad,), n_rel, jnp.int32)])
    head_p = jnp.concatenate([head, jnp.full((pad,), N_PAD - 8, jnp.int32)])
    edata = jnp.stack([tail_p.reshape(NW, nb, EB),
                       etype_p.reshape(NW, nb, EB),
                       head_p.reshape(NW, nb, EB)], axis=2)

    relp = jnp.zeros((NREL_PAD, CH), jnp.float32).at[:n_rel].set(relation_emb)
    zrow = jnp.zeros((ROWS_PER_TILE, CH), jnp.bfloat16)
    zcnt = jnp.zeros((ROWS_PER_TILE, CNT_W), jnp.float32)
    ones = jnp.ones((EB, CNT_W), jnp.float32)

    sc_agg = _make_sc_agg(nb)

    reln, relres = _tc_rel(relp)
    rel1 = relp.astype(jnp.bfloat16)

    res0 = jnp.zeros((N_PAD, CH), jnp.float32).at[:n_ent].set(entity_emb)

    def hop(carry, rel_tab):
        ent_tab, res = carry
        part, cntp = sc_agg(ent_tab, rel_tab, edata, zrow, zcnt, ones)
        ent_next, res_next = _tc_hop(part, cntp, res)
        return (ent_next, res_next), None

    rel_tabs = jnp.stack([rel1, reln])
    (_, res2), _ = lax.scan(hop, (res0.astype(jnp.bfloat16), res0), rel_tabs)

    drug0p = jnp.zeros((N_PAD, CH), jnp.float32).at[:n_drugs].set(drug_emb)
    drug_full = _tc_drug(res2, drug0p, res0)

    entity_res = res2[:n_ent]
    drug_res = drug_full[:n_drugs]
    relation_res = relres[:n_rel]
    return (entity_res, drug_res, relation_res)


# speedup vs baseline: 1.1601x; 1.1601x over previous
import functools

import jax
import jax.numpy as jnp
from jax import lax
from jax.experimental import pallas as pl
from jax.experimental.pallas import tpu as pltpu
from jax.experimental.pallas import tpu_sc as plsc

CH = 128
L = 16
BL = 32
NC = 2
NS = 16
NW = NC * NS
EB = 128
N_PAD = 10240
ROWS_PER_TILE = N_PAD // NS
CNT_W = 16
NREL_PAD = 16
RING = 3
IR = 2 * RING


def _make_sc_agg(nb):
    assert nb % IR == 0
    out_type = [
        jax.ShapeDtypeStruct((NC, N_PAD, CH), jnp.bfloat16),
        jax.ShapeDtypeStruct((NC, N_PAD, CNT_W), jnp.float32),
    ]
    scratch = [
        pltpu.VMEM((IR, 3, EB), jnp.int32),
        pltpu.VMEM((RING, EB, CH), jnp.bfloat16),
        pltpu.VMEM((RING, EB, CH), jnp.bfloat16),
        pltpu.VMEM((NREL_PAD, CH), jnp.bfloat16),
        pltpu.VMEM((EB, CNT_W), jnp.float32),
        pltpu.VMEM_SHARED((N_PAD, CH), jnp.bfloat16),
        pltpu.VMEM_SHARED((N_PAD, CNT_W), jnp.float32),
    ]
    scratch += [pltpu.SemaphoreType.DMA] * RING
    scratch += [pltpu.SemaphoreType.DMA] * IR
    mesh = plsc.VectorSubcoreMesh(core_axis_name="c", subcore_axis_name="s")

    @functools.partial(pl.kernel, out_type=tuple(out_type), mesh=mesh,
                       scratch_types=scratch,
                       compiler_params=pltpu.CompilerParams(
                           use_tc_tiling_on_sc=False))
    def sc_agg(ent_hbm, rel_hbm, edata_hbm, zrow_hbm, zcnt_hbm, ones_hbm,
               *rest):
        (part_hbm, cnt_hbm, eslot, erow, msg, reltab, ones_v, acc, cacc,
         *sems) = rest
        sem_g = sems[:RING]
        sem_i = sems[RING:]
        c = lax.axis_index("c")
        s = lax.axis_index("s")
        wid = c * NS + s
        pltpu.sync_copy(rel_hbm, reltab)
        tile_rows = pl.ds(s * ROWS_PER_TILE, ROWS_PER_TILE)
        pltpu.sync_copy(zrow_hbm, acc.at[tile_rows])
        pltpu.sync_copy(zcnt_hbm, cacc.at[tile_rows])
        pltpu.sync_copy(ones_hbm, ones_v)
        plsc.subcore_barrier()

        def fire_idx(u, b):
            pltpu.async_copy(edata_hbm.at[wid, b], eslot.at[u], sem_i[u])

        def wait_idx(u):
            pltpu.make_async_copy(edata_hbm.at[0, 0], eslot.at[u],
                                  sem_i[u]).wait()

        def fire_row(k, u):
            pltpu.async_copy(ent_hbm.at[eslot.at[u, 0]], erow.at[k],
                             sem_g[k])

        def wait_row(k):
            pltpu.make_async_copy(ent_hbm.at[eslot.at[0, 0]],
                                  erow.at[k], sem_g[k]).wait()

        for u in range(IR):
            fire_idx(u, u)
        for k in range(RING):
            wait_idx(k)
            fire_row(k, k)

        def group_body(g, carry):
            for u in range(IR):
                k = u % RING
                b = g * IR + u
                wait_row(k)

                def mul_body(g2, carry2):
                    tvec = eslot[u, 1, pl.ds(g2 * L, L)]
                    for i in range(L):
                        e = g2 * L + i
                        t = tvec[i]
                        for j in range(CH // BL):
                            sl = pl.ds(j * BL, BL)
                            msg[k, e, sl] = erow[k, e, sl] * reltab[t, sl]
                    return carry2

                lax.fori_loop(0, EB // L, mul_body, 0)

                pltpu.sync_copy(msg.at[k], acc.at[eslot.at[u, 2]], add=True)
                pltpu.sync_copy(ones_v, cacc.at[eslot.at[u, 2]], add=True)

                @pl.when(b + IR < nb)
                def _():
                    fire_idx(u, b + IR)

                @pl.when(b + RING < nb)
                def _():
                    u2 = (u + RING) % IR
                    wait_idx(u2)
                    fire_row(k, u2)
            return carry

        lax.fori_loop(0, nb // IR, group_body, 0)
        plsc.subcore_barrier()
        pltpu.sync_copy(acc.at[tile_rows], part_hbm.at[c, tile_rows])
        pltpu.sync_copy(cacc.at[tile_rows], cnt_hbm.at[c, tile_rows])

    return sc_agg


TBLK = 1280


def _tc_hop(part, cntp, res_in):
    def body(part_ref, cntp_ref, res_ref, ent_next_ref, res_out_ref):
        ssum = (part_ref[0].astype(jnp.float32)
                + part_ref[1].astype(jnp.float32))
        cnt = cntp_ref[0, :, 0:1] + cntp_ref[1, :, 0:1]
        mean = ssum / jnp.maximum(cnt, 1.0)
        nrm = jnp.sqrt(jnp.sum(mean * mean, axis=1, keepdims=True))
        normd = mean / jnp.maximum(nrm, 1e-12)
        ent_next_ref[...] = normd.astype(jnp.bfloat16)
        res_out_ref[...] = res_ref[...] + normd

    grid = N_PAD // TBLK
    return pl.pallas_call(
        body,
        grid=(grid,),
        in_specs=[
            pl.BlockSpec((NC, TBLK, CH), lambda i: (0, i, 0)),
            pl.BlockSpec((NC, TBLK, CNT_W), lambda i: (0, i, 0)),
            pl.BlockSpec((TBLK, CH), lambda i: (i, 0)),
        ],
        out_specs=[
            pl.BlockSpec((TBLK, CH), lambda i: (i, 0)),
            pl.BlockSpec((TBLK, CH), lambda i: (i, 0)),
        ],
        out_shape=[
            jax.ShapeDtypeStruct((N_PAD, CH), jnp.bfloat16),
            jax.ShapeDtypeStruct((N_PAD, CH), jnp.float32),
        ],
    )(part, cntp, res_in)


def _tc_rel(relp):
    def body(rel_ref, reln_ref, relres_ref):
        r = rel_ref[...]
        nrm = jnp.sqrt(jnp.sum(r * r, axis=1, keepdims=True))
        rn = r / jnp.maximum(nrm, 1e-12)
        reln_ref[...] = rn.astype(jnp.bfloat16)
        relres_ref[...] = r + 2.0 * rn

    return pl.pallas_call(
        body,
        out_shape=[
            jax.ShapeDtypeStruct((NREL_PAD, CH), jnp.bfloat16),
            jax.ShapeDtypeStruct((NREL_PAD, CH), jnp.float32),
        ],
    )(relp)


def _tc_cast(x):
    def body(x_ref, o_ref):
        o_ref[...] = x_ref[...].astype(jnp.bfloat16)

    grid = N_PAD // TBLK
    return pl.pallas_call(
        body,
        grid=(grid,),
        in_specs=[pl.BlockSpec((TBLK, CH), lambda i: (i, 0))],
        out_specs=[pl.BlockSpec((TBLK, CH), lambda i: (i, 0))],
        out_shape=[jax.ShapeDtypeStruct((N_PAD, CH), jnp.bfloat16)],
    )(x)[0]


def _tc_drug(res2, drug0p, ent0p):
    def body(res_ref, drug0_ref, ent0_ref, drug_ref):
        drug_ref[...] = drug0_ref[...] + (res_ref[...] - ent0_ref[...])

    grid = N_PAD // TBLK
    return pl.pallas_call(
        body,
        grid=(grid,),
        in_specs=[
            pl.BlockSpec((TBLK, CH), lambda i: (i, 0)),
            pl.BlockSpec((TBLK, CH), lambda i: (i, 0)),
            pl.BlockSpec((TBLK, CH), lambda i: (i, 0)),
        ],
        out_specs=[
            pl.BlockSpec((TBLK, CH), lambda i: (i, 0)),
        ],
        out_shape=[
            jax.ShapeDtypeStruct((N_PAD, CH), jnp.float32),
        ],
    )(res2, drug0p, ent0p)[0]


def kernel(drug_emb, entity_emb, relation_emb, edge_index, edge_type, gpu_id):
    n_ent, _ = entity_emb.shape
    n_drugs = drug_emb.shape[0]
    n_rel = relation_emb.shape[0]
    e = edge_type.shape[0]

    head = edge_index[0].astype(jnp.int32)
    tail = edge_index[1].astype(jnp.int32)
    etype = edge_type.astype(jnp.int32)

    nb = IR * -(-e // (NW * EB * IR))
    e_pad = NW * nb * EB
    pad = e_pad - e
    tail_p = jnp.concatenate([tail, jnp.zeros((pad,), jnp.int32)])
    etype_p = jnp.concatenate([etype, jnp.full((pad,), n_rel, jnp.int32)])
    head_p = jnp.concatenate([head, jnp.full((pad,), N_PAD - 8, jnp.int32)])
    edata = jnp.stack([tail_p.reshape(NW, nb, EB),
                       etype_p.reshape(NW, nb, EB),
                       head_p.reshape(NW, nb, EB)], axis=2)

    relp = jnp.zeros((NREL_PAD, CH), jnp.float32).at[:n_rel].set(relation_emb)
    zrow = jnp.zeros((ROWS_PER_TILE, CH), jnp.bfloat16)
    zcnt = jnp.zeros((ROWS_PER_TILE, CNT_W), jnp.float32)
    ones = jnp.ones((EB, CNT_W), jnp.float32)

    sc_agg = _make_sc_agg(nb)

    reln, relres = _tc_rel(relp)
    rel1 = relp.astype(jnp.bfloat16)

    res0 = jnp.zeros((N_PAD, CH), jnp.float32).at[:n_ent].set(entity_emb)

    def hop(carry, rel_tab):
        ent_tab, res = carry
        part, cntp = sc_agg(ent_tab, rel_tab, edata, zrow, zcnt, ones)
        ent_next, res_next = _tc_hop(part, cntp, res)
        return (ent_next, res_next), None

    rel_tabs = jnp.stack([rel1, reln])
    (_, res2), _ = lax.scan(hop, (_tc_cast(res0), res0), rel_tabs)

    drug0p = jnp.zeros((N_PAD, CH), jnp.float32).at[:n_drugs].set(drug_emb)
    drug_full = _tc_drug(res2, drug0p, res0)

    entity_res = res2[:n_ent]
    drug_res = drug_full[:n_drugs]
    relation_res = relres[:n_rel]
    return (entity_res, drug_res, relation_res)
